# baseline (speedup 1.0000x reference)
import jax
import jax.numpy as jnp
from jax import lax
from jax.experimental import pallas as pl
from jax.experimental.pallas import tpu as pltpu
from jax.experimental.pallas import tpu_sc as plsc

B = 128
N = 32768
K = 16
L = 16
NC = 2
NS = 16
NW = NC * NS
ROWS_PER_W = B // NW

NEG = float("-inf")
BIGI = 2**30
CAND_CAP = 1024

G = 8
NG = N // (L * G)
_PASS_B_GRP = 4


def _vmax_scalar(x):
    return plsc.cummax(x)[L - 1]


def _vmin_scalar(x):
    return -plsc.cummax(-x)[L - 1]


def _topk_body(scores_hbm, out_hbm, row_v, outrow_v, gmax, cvals, cidxs):
    wid = lax.axis_index("c") * NS + lax.axis_index("s")
    lane = lax.iota(jnp.int32, L)

    @plsc.parallel_loop(0, N // L)
    def _zero(i):
        outrow_v[pl.ds(i * L, L)] = jnp.zeros((L,), jnp.float32)

    for rr in range(ROWS_PER_W):
        row = wid * ROWS_PER_W + rr
        pltpu.sync_copy(scores_hbm.at[row], row_v)

        @plsc.parallel_loop(0, NG, unroll=2)
        def _pass_a(j):
            base = j * (L * G)
            vs = [row_v[pl.ds(base + k * L, L)] for k in range(G)]
            while len(vs) > 1:
                vs = [jnp.maximum(vs[p], vs[p + 1])
                      for p in range(0, len(vs), 2)]
            gmax[pl.ds(j * L, L)] = vs[0]

        accs0 = tuple(jnp.full((L,), NEG) for _ in range(8))

        @plsc.parallel_loop(0, NG // 8, carry=accs0)
        def _fold(i, accs):
            base = i * 8 * L
            return tuple(
                jnp.maximum(a, gmax[pl.ds(base + k * L, L)])
                for k, a in enumerate(accs)
            )

        m = _fold[0]
        for a in _fold[1:]:
            m = jnp.maximum(m, a)
        t0 = _vmin_scalar(m)

        @plsc.parallel_loop(0, NG // _PASS_B_GRP, carry=jnp.int32(0))
        def _pass_b(i, off):
            gb = i * _PASS_B_GRP
            gs = [gmax[pl.ds((gb + k) * L, L)] for k in range(_PASS_B_GRP)]
            ms = [g >= t0 for g in gs]
            anym = ms[0]
            for mk in ms[1:]:
                anym = anym | mk

            def slow(off):
                for k in range(_PASS_B_GRP):
                    def scan_group(off, k=k):
                        base = (gb + k) * (L * G)
                        for q in range(G):
                            v = row_v[pl.ds(base + q * L, L)]
                            mk = v >= t0
                            cnt = plsc.all_reduce_population_count(mk)[0]
                            plsc.store_compressed(
                                cvals.at[pl.ds(off, L)], v, mask=mk)
                            plsc.store_compressed(
                                cidxs.at[pl.ds(off, L)],
                                lane + (base + q * L), mask=mk)
                            off = jnp.minimum(off + cnt, CAND_CAP)
                        return off

                    hask = plsc.all_reduce_population_count(ms[k])[0] > 0
                    off = lax.cond(hask, scan_group, lambda o: o, off)
                return off

            have = plsc.all_reduce_population_count(anym)[0] > 0
            return lax.cond(have, slow, lambda o: o, off)

        ncand = _pass_b
        nv = (ncand + (L - 1)) // L

        def _clean(j, _):
            pos = lane + j * L
            v = cvals[pl.ds(j * L, L)]
            cvals[pl.ds(j * L, L)] = jnp.where(pos < ncand, v, NEG)
            return 0

        lax.fori_loop(nv - 1, nv, _clean, 0)

        def _round(r, selvec):
            def scan(j, st):
                bv, bi = st
                v = cvals[pl.ds(j * L, L)]
                ix = cidxs[pl.ds(j * L, L)]
                take = (v > bv) | ((v == bv) & (ix < bi))
                return (jnp.where(take, v, bv), jnp.where(take, ix, bi))

            bv, bi = lax.fori_loop(
                0, nv, scan,
                (jnp.full((L,), NEG), jnp.full((L,), BIGI)))
            mval = _vmax_scalar(bv)
            sel = _vmin_scalar(jnp.where(bv == mval, bi, BIGI))

            def suppress(j, _):
                v = cvals[pl.ds(j * L, L)]
                ix = cidxs[pl.ds(j * L, L)]
                cvals[pl.ds(j * L, L)] = jnp.where(ix == sel, NEG, v)
                return 0

            lax.fori_loop(0, nv, suppress, 0)
            return jnp.where(lane == r, sel, selvec)

        selvec = lax.fori_loop(0, K, _round, jnp.full((L,), BIGI))

        plsc.store_scatter(outrow_v, [selvec], jnp.ones((L,), jnp.float32))
        pltpu.sync_copy(outrow_v, out_hbm.at[row])
        plsc.store_scatter(outrow_v, [selvec], jnp.zeros((L,), jnp.float32))


@jax.jit
def _topk_mask(scores):
    mesh = plsc.VectorSubcoreMesh(
        core_axis_name="c", subcore_axis_name="s")
    return pl.kernel(
        _topk_body,
        out_type=jax.ShapeDtypeStruct((B, N), jnp.float32),
        mesh=mesh,
        compiler_params=pltpu.CompilerParams(needs_layout_passes=False),
        scratch_types=[
            pltpu.VMEM((N,), jnp.float32),
            pltpu.VMEM((N,), jnp.float32),
            pltpu.VMEM((NG * L,), jnp.float32),
            pltpu.VMEM((CAND_CAP + L,), jnp.float32),
            pltpu.VMEM((CAND_CAP + L,), jnp.int32),
        ],
    )(scores)


def kernel(scores):
    return _topk_mask(scores)


# speedup vs baseline: 1.3163x; 1.3163x over previous
import jax
import jax.numpy as jnp
from jax import lax
from jax.experimental import pallas as pl
from jax.experimental.pallas import tpu as pltpu
from jax.experimental.pallas import tpu_sc as plsc

B = 128
N = 32768
K = 16
L = 16
NC = 2
NS = 16
NW = NC * NS
ROWS_PER_W = B // NW

NEG = float("-inf")
BIGI = 2**30
CAND_CAP = 1024

G = 8
NG = N // (L * G)
_PASS_B_GRP = 4


def _sorted_desc(v):
    k, _ = plsc.sort_key_val(v, v, descending=True)
    return k


def _merge_top16(a, b):
    m = jnp.maximum(a, lax.rev(b, (0,)))
    return _sorted_desc(m)


def _topk_body(scores_hbm, out_hbm, row_v, outrow_v, gmax, cvals, cidxs,
               selbuf):
    wid = lax.axis_index("c") * NS + lax.axis_index("s")
    lane = lax.iota(jnp.int32, L)

    @plsc.parallel_loop(0, N // L)
    def _zero(i):
        outrow_v[pl.ds(i * L, L)] = jnp.zeros((L,), jnp.float32)

    for rr in range(ROWS_PER_W):
        row = wid * ROWS_PER_W + rr
        pltpu.sync_copy(scores_hbm.at[row], row_v)

        @plsc.parallel_loop(0, NG, unroll=2)
        def _pass_a(j):
            base = j * (L * G)
            vs = [row_v[pl.ds(base + k * L, L)] for k in range(G)]
            while len(vs) > 1:
                vs = [jnp.maximum(vs[p], vs[p + 1])
                      for p in range(0, len(vs), 2)]
            gmax[pl.ds(j * L, L)] = vs[0]

        accs0 = tuple(jnp.full((L,), NEG) for _ in range(8))

        @plsc.parallel_loop(0, NG // 8, carry=accs0)
        def _fold(i, accs):
            base = i * 8 * L
            return tuple(
                jnp.maximum(a, gmax[pl.ds(base + k * L, L)])
                for k, a in enumerate(accs)
            )

        srt = [_sorted_desc(a) for a in _fold]
        while len(srt) > 1:
            srt = [_merge_top16(srt[p], srt[p + 1])
                   for p in range(0, len(srt), 2)]
        t0 = srt[0][L - 1]

        @plsc.parallel_loop(0, NG // _PASS_B_GRP, carry=jnp.int32(0))
        def _pass_b(i, off):
            gb = i * _PASS_B_GRP
            gs = [gmax[pl.ds((gb + k) * L, L)] for k in range(_PASS_B_GRP)]
            ms = [g >= t0 for g in gs]
            anym = ms[0]
            for mk in ms[1:]:
                anym = anym | mk

            def slow(off):
                for k in range(_PASS_B_GRP):
                    def scan_group(off, k=k):
                        base = (gb + k) * (L * G)
                        for q in range(G):
                            v = row_v[pl.ds(base + q * L, L)]
                            mk = v >= t0
                            cnt = plsc.all_reduce_population_count(mk)[0]
                            plsc.store_compressed(
                                cvals.at[pl.ds(off, L)], v, mask=mk)
                            plsc.store_compressed(
                                cidxs.at[pl.ds(off, L)],
                                lane + (base + q * L), mask=mk)
                            off = jnp.minimum(off + cnt, CAND_CAP)
                        return off

                    hask = plsc.all_reduce_population_count(ms[k])[0] > 0
                    off = lax.cond(hask, scan_group, lambda o: o, off)
                return off

            have = plsc.all_reduce_population_count(anym)[0] > 0
            return lax.cond(have, slow, lambda o: o, off)

        ncand = _pass_b
        nv = (ncand + (L - 1)) // L

        def _clean(j, _):
            pos = lane + j * L
            v = cvals[pl.ds(j * L, L)]
            cvals[pl.ds(j * L, L)] = jnp.where(pos < ncand, v, NEG)
            return 0

        lax.fori_loop(nv - 1, nv, _clean, 0)

        def _tfold(j, cur):
            return _merge_top16(cur, _sorted_desc(cvals[pl.ds(j * L, L)]))

        top16 = lax.fori_loop(0, nv, _tfold, jnp.full((L,), NEG))
        t = top16[L - 1]

        def _px(j, off):
            v = cvals[pl.ds(j * L, L)]
            ix = cidxs[pl.ds(j * L, L)]
            mk = v > t
            plsc.store_compressed(selbuf.at[pl.ds(off, L)], ix, mask=mk)
            return off + plsc.all_reduce_population_count(mk)[0]

        g = lax.fori_loop(0, nv, _px, jnp.int32(0))

        def _py(j, off):
            v = cvals[pl.ds(j * L, L)]
            ix = cidxs[pl.ds(j * L, L)]
            mk = v == t
            plsc.store_compressed(selbuf.at[pl.ds(off, L)], ix, mask=mk)
            return off + plsc.all_reduce_population_count(mk)[0]

        lax.fori_loop(0, nv, _py, g)
        selvec = selbuf[pl.ds(0, L)]

        plsc.store_scatter(outrow_v, [selvec], jnp.ones((L,), jnp.float32))
        pltpu.sync_copy(outrow_v, out_hbm.at[row])
        plsc.store_scatter(outrow_v, [selvec], jnp.zeros((L,), jnp.float32))


@jax.jit
def _topk_mask(scores):
    mesh = plsc.VectorSubcoreMesh(
        core_axis_name="c", subcore_axis_name="s")
    return pl.kernel(
        _topk_body,
        out_type=jax.ShapeDtypeStruct((B, N), jnp.float32),
        mesh=mesh,
        compiler_params=pltpu.CompilerParams(needs_layout_passes=False),
        scratch_types=[
            pltpu.VMEM((N,), jnp.float32),
            pltpu.VMEM((N,), jnp.float32),
            pltpu.VMEM((NG * L,), jnp.float32),
            pltpu.VMEM((CAND_CAP + L,), jnp.float32),
            pltpu.VMEM((CAND_CAP + L,), jnp.int32),
            pltpu.VMEM((CAND_CAP + L,), jnp.int32),
        ],
    )(scores)


def kernel(scores):
    return _topk_mask(scores)
